# baseline (speedup 1.0000x reference)
import functools

import jax
import jax.numpy as jnp
from jax import lax
from jax.experimental import pallas as pl
from jax.experimental.pallas import tpu as pltpu
from jax.experimental.pallas import tpu_sc as plsc

_NUM_CORES = 2
_NUM_SUBCORES = 16
_NUM_WORKERS = _NUM_CORES * _NUM_SUBCORES
_CHUNK = 32


@functools.lru_cache(maxsize=None)
def _make_gather(n, e):
    per_w = n // _NUM_WORKERS
    assert per_w * _NUM_WORKERS == n and per_w % (2 * _CHUNK) == 0
    npair = per_w // (2 * _CHUNK)
    mesh = plsc.VectorSubcoreMesh(core_axis_name="c", subcore_axis_name="s")

    @functools.partial(
        pl.kernel,
        mesh=mesh,
        out_type=jax.ShapeDtypeStruct((n, e), jnp.float32),
        scratch_types=[
            pltpu.VMEM((per_w,), jnp.int32),
            pltpu.VMEM((8 * _CHUNK, e), jnp.float32),
            pltpu.VMEM((8 * _CHUNK, e), jnp.float32),
            pltpu.VMEM((_CHUNK, e), jnp.float32),
            pltpu.SemaphoreType.DMA,
            pltpu.SemaphoreType.DMA,
            pltpu.SemaphoreType.DMA,
        ],
        compiler_params=pltpu.CompilerParams(use_tc_tiling_on_sc=True),
    )
    def gather(tbl_hbm, idx_hbm, out_hbm, idx_v, buf0, buf1, crow, sem_i, sem0, sem1):
        wid = lax.axis_index("s") * _NUM_CORES + lax.axis_index("c")
        base = wid * per_w
        pltpu.async_copy(idx_hbm.at[pl.ds(base, per_w)], idx_v, sem_i).wait()

        def fire(c, buf, sem):
            for g in range(_CHUNK // 16):
                vec = idx_v[pl.ds(c * _CHUNK + g * 16, 16)]
                for j in range(16):
                    v = vec[j]
                    k = g * 16 + j
                    pltpu.async_copy(
                        tbl_hbm.at[v // 8],
                        buf.at[pl.ds(k * 8, 8), :],
                        sem,
                    )

        def drain(buf, sem):
            for k in range(_CHUNK):
                pltpu.make_async_copy(
                    tbl_hbm.at[0],
                    buf.at[pl.ds(k * 8, 8), :],
                    sem,
                ).wait()

        def select_writeback(c, buf):
            for g in range(_CHUNK // 16):
                vec = idx_v[pl.ds(c * _CHUNK + g * 16, 16)]
                for j in range(16):
                    v = vec[j]
                    r = v - (v // 8) * 8
                    k = g * 16 + j
                    for l in range(e // 16):
                        crow[k, pl.ds(16 * l, 16)] = buf[k * 8 + r, pl.ds(16 * l, 16)]
            off = pl.multiple_of(base + c * _CHUNK, 8)
            pltpu.sync_copy(crow, out_hbm.at[pl.ds(off, _CHUNK), :])

        fire(0, buf0, sem0)

        def pair(p):
            c0 = 2 * p
            fire(c0 + 1, buf1, sem1)
            drain(buf0, sem0)
            select_writeback(c0, buf0)

            @pl.when(p + 1 < npair)
            def _next_even():
                fire(c0 + 2, buf0, sem0)

            drain(buf1, sem1)
            select_writeback(c0 + 1, buf1)
            return None

        pl.loop(0, npair)(pair)

    return gather


@functools.lru_cache(maxsize=None)
def _make_lstm(seq_len, b, e, h):
    g4 = 4 * h

    def body(xt_ref, emb_ref, wih_ref, whh_ref, b_ref, h_ref, c_ref):
        t = pl.program_id(0)

        @pl.when(t == 0)
        def _init():
            h_ref[...] = jnp.zeros_like(h_ref)
            c_ref[...] = jnp.zeros_like(c_ref)

        mask = (xt_ref[0, 0, :] != 0).astype(jnp.float32)
        xt = emb_ref[0] * mask[:, None]
        gates = (
            jnp.dot(xt, wih_ref[...], preferred_element_type=jnp.float32)
            + jnp.dot(h_ref[...], whh_ref[...], preferred_element_type=jnp.float32)
            + b_ref[...]
        )
        i = jax.nn.sigmoid(gates[:, 0:h])
        f = jax.nn.sigmoid(gates[:, h:2 * h])
        g = jnp.tanh(gates[:, 2 * h:3 * h])
        o = jax.nn.sigmoid(gates[:, 3 * h:4 * h])
        c = f * c_ref[...] + i * g
        c_ref[...] = c
        h_ref[...] = o * jnp.tanh(c)

    return pl.pallas_call(
        body,
        grid=(seq_len,),
        in_specs=[
            pl.BlockSpec((1, 1, b), lambda t: (t, 0, 0)),
            pl.BlockSpec((1, b, e), lambda t: (t, 0, 0)),
            pl.BlockSpec((e, g4), lambda t: (0, 0)),
            pl.BlockSpec((h, g4), lambda t: (0, 0)),
            pl.BlockSpec((1, g4), lambda t: (0, 0)),
        ],
        out_specs=[
            pl.BlockSpec((b, h), lambda t: (0, 0)),
            pl.BlockSpec((b, h), lambda t: (0, 0)),
        ],
        out_shape=[
            jax.ShapeDtypeStruct((b, h), jnp.float32),
            jax.ShapeDtypeStruct((b, h), jnp.float32),
        ],
    )


def kernel(x, table, W_ih, W_hh, b_ih, b_hh):
    b, seq_len = x.shape
    e = table.shape[1]
    h = W_hh.shape[1]
    n = seq_len * b
    idx = x.T.reshape(-1)
    emb = _make_gather(n, e)(table.reshape(-1, 8, e), idx)
    emb = emb.reshape(seq_len, b, e)
    x_tm = x.T.reshape(seq_len, 1, b)
    bias = (b_ih + b_hh).reshape(1, 4 * h)
    hN, cN = _make_lstm(seq_len, b, e, h)(x_tm, emb, W_ih.T, W_hh.T, bias)
    return (hN[None], cN[None])
